# baseline (speedup 1.0000x reference)
import functools

import jax
import jax.numpy as jnp
from jax import lax
from jax.experimental import pallas as pl
from jax.experimental.pallas import tpu as pltpu
from jax.experimental.pallas import tpu_sc as plsc

_N = 10000
_E = 320000
_D = 128
_NP = 10240
_NC, _NS = 2, 16
_NW = _NC * _NS
_EPW = _E // _NW
_EPWP = 10240
_EP = _EPWP * _NW
_K = 80
_NCH = _EPWP // _K
_NB = 1
_DL = 1
_G = 64
_NSEG = _NCH // _G
_RPT = _NP // _NS
_B = 512
_GRID = _NP // _B

_mesh = plsc.VectorSubcoreMesh(core_axis_name="c", subcore_axis_name="s")


@functools.partial(
    pl.kernel,
    out_type=jax.ShapeDtypeStruct((_NW, _NP), jnp.float32),
    mesh=_mesh,
    scratch_types=[
        pltpu.VMEM((_EPW,), jnp.int32),
        pltpu.VMEM((_NP,), jnp.float32),
    ],
    compiler_params=pltpu.CompilerParams(needs_layout_passes=False),
)
def _sc_degree(dst_hbm, degp_hbm, dbuf, hist):
    wid = lax.axis_index("c") * _NS + lax.axis_index("s")
    pltpu.sync_copy(dst_hbm.at[pl.ds(wid * _EPW, _EPW)], dbuf)
    zeros = jnp.zeros((16,), jnp.float32)

    def zbody(i, _):
        hist[pl.ds(i * 16, 16)] = zeros
        return 0

    lax.fori_loop(0, _NP // 16, zbody, 0)
    ones = jnp.ones((16,), jnp.float32)

    def body(i, _):
        idx = dbuf[pl.ds(i * 16, 16)]
        plsc.addupdate_scatter(hist, [idx], ones)
        return 0

    lax.fori_loop(0, _EPW // 16, body, 0)
    pltpu.sync_copy(hist, degp_hbm.at[wid])


@functools.partial(
    pl.kernel,
    out_type=jax.ShapeDtypeStruct((_NC, _NP, _D), jnp.float32),
    mesh=_mesh,
    scratch_types=[
        pltpu.VMEM((2, _G, _K), jnp.int32),
        pltpu.VMEM((_K,), jnp.int32),
        pltpu.VMEM((_K,), jnp.int32),
        pltpu.VMEM((_NB, _K, _D), jnp.float32),
        pltpu.VMEM_SHARED((_NP, _D), jnp.float32),
        pltpu.SemaphoreType.DMA((_NB,)),
        pltpu.SemaphoreType.DMA((_NB,)),
    ],
)
def _sc_pass(e5_hbm, xt_hbm, aggp_hbm, ibuf, sidx, didx, rows, acc, gsem,
             ssem):
    cid = lax.axis_index("c")
    sid = lax.axis_index("s")
    wid = cid * _NS + sid

    zeros = jnp.zeros((16,), jnp.float32)

    def zb(i, _):
        rows[0, i // 8, pl.ds((i % 8) * 16, 16)] = zeros
        return 0

    lax.fori_loop(0, _K * 8, zb, 0)

    def za(i, _):
        pltpu.sync_copy(rows.at[0], acc.at[pl.ds(sid * _RPT + i * _K, _K)])
        return 0

    lax.fori_loop(0, _RPT // _K, za, 0)
    plsc.subcore_barrier()

    def g_issue(g, b):
        pltpu.async_copy(xt_hbm.at[ibuf.at[0, g]], rows.at[b], gsem.at[b])

    def g_wait(g, b):
        pltpu.make_async_copy(xt_hbm.at[ibuf.at[0, g]], rows.at[b],
                              gsem.at[b]).wait()

    def s_issue(g, b):
        pltpu.async_copy(rows.at[b], acc.at[ibuf.at[1, g]], ssem.at[b],
                         add=True)

    def s_wait(g, b):
        pltpu.make_async_copy(rows.at[b], acc.at[ibuf.at[1, g]],
                              ssem.at[b]).wait()

    def segment(s, _):
        pltpu.sync_copy(e5_hbm.at[wid, s], ibuf)
        for g in range(_G):
            for c in range(_K // 16):
                sidx[pl.ds(c * 16, 16)] = ibuf[0, g, pl.ds(c * 16, 16)]
                didx[pl.ds(c * 16, 16)] = ibuf[1, g, pl.ds(c * 16, 16)]
            pltpu.async_copy(xt_hbm.at[sidx], rows.at[0], gsem.at[0]).wait()
            pltpu.sync_copy(rows.at[0], acc.at[didx], add=True)
        return 0

    lax.fori_loop(0, _NSEG, segment, 0)

    plsc.subcore_barrier()
    pltpu.sync_copy(
        acc.at[pl.ds(sid * _RPT, _RPT)],
        aggp_hbm.at[cid, pl.ds(sid * _RPT, _RPT)],
    )


def _dinv(degp):
    ones = jnp.ones((_NW, 1), jnp.float32)
    deg = lax.dot_general(degp, ones, (((0,), (0,)), ((), ())))
    return lax.rsqrt(jnp.maximum(deg, 1.0))


def _tc_pre_body(x_ref, w1_ref, degp_ref, xt1_ref):
    xw = jnp.dot(x_ref[...], w1_ref[...], preferred_element_type=jnp.float32)
    xt1_ref[...] = xw * _dinv(degp_ref[...])


def _tc_mid_body(aggp_ref, degp_ref, b1_ref, xt2_ref):
    di = _dinv(degp_ref[...])
    h1 = jnp.maximum(di * (aggp_ref[0] + aggp_ref[1]) + b1_ref[...], 0.0)
    xt2_ref[...] = di * h1


def _tc_final_body(aggp_ref, degp_ref, w2_ref, wl1_ref, b2_ref, bl1_ref,
                   wl2_ref, bl2_ref, out_ref, acc, w25, b25):
    i = pl.program_id(0)

    @pl.when(i == 0)
    def _():
        w25[...] = jnp.dot(w2_ref[...], wl1_ref[...],
                           preferred_element_type=jnp.float32)
        b25[...] = jnp.dot(b2_ref[...], wl1_ref[...],
                           preferred_element_type=jnp.float32) + bl1_ref[...]
        acc[...] = jnp.zeros((1, _D), jnp.float32)

    a2 = _dinv(degp_ref[...]) * (aggp_ref[0] + aggp_ref[1])
    h = jnp.maximum(
        jnp.dot(a2, w25[...], preferred_element_type=jnp.float32) + b25[...],
        0.0,
    )
    row = i * _B + lax.broadcasted_iota(jnp.int32, (_B, 1), 0)
    h = jnp.where(row < _N, h, 0.0)
    acc[...] += jnp.sum(h, axis=0, keepdims=True)

    @pl.when(i == _GRID - 1)
    def _():
        g = acc[...] * (1.0 / _N)
        out_ref[...] = jnp.dot(g, wl2_ref[...],
                               preferred_element_type=jnp.float32) + bl2_ref[...]


def _tc_pre(x_pad, W1, degp):
    return pl.pallas_call(
        _tc_pre_body,
        grid=(_GRID,),
        in_specs=[
            pl.BlockSpec((_B, _D), lambda i: (i, 0)),
            pl.BlockSpec((_D, _D), lambda i: (0, 0)),
            pl.BlockSpec((_NW, _B), lambda i: (0, i)),
        ],
        out_specs=pl.BlockSpec((_B, _D), lambda i: (i, 0)),
        out_shape=jax.ShapeDtypeStruct((_NP, _D), jnp.float32),
    )(x_pad, W1, degp)


def _tc_mid(aggp, degp, b1r):
    return pl.pallas_call(
        _tc_mid_body,
        grid=(_GRID,),
        in_specs=[
            pl.BlockSpec((_NC, _B, _D), lambda i: (0, i, 0)),
            pl.BlockSpec((_NW, _B), lambda i: (0, i)),
            pl.BlockSpec((1, _D), lambda i: (0, 0)),
        ],
        out_specs=pl.BlockSpec((_B, _D), lambda i: (i, 0)),
        out_shape=jax.ShapeDtypeStruct((_NP, _D), jnp.float32),
    )(aggp, degp, b1r)


def _tc_final(aggp, degp, W2, Wl1, b2r, bl1r, Wl2, bl2r):
    return pl.pallas_call(
        _tc_final_body,
        grid=(_GRID,),
        in_specs=[
            pl.BlockSpec((_NC, _B, _D), lambda i: (0, i, 0)),
            pl.BlockSpec((_NW, _B), lambda i: (0, i)),
            pl.BlockSpec((_D, _D), lambda i: (0, 0)),
            pl.BlockSpec((_D, _D), lambda i: (0, 0)),
            pl.BlockSpec((1, _D), lambda i: (0, 0)),
            pl.BlockSpec((1, _D), lambda i: (0, 0)),
            pl.BlockSpec((_D, _D), lambda i: (0, 0)),
            pl.BlockSpec((1, _D), lambda i: (0, 0)),
        ],
        out_specs=pl.BlockSpec((1, _D), lambda i: (0, 0)),
        out_shape=jax.ShapeDtypeStruct((1, _D), jnp.float32),
        scratch_shapes=[
            pltpu.VMEM((1, _D), jnp.float32),
            pltpu.VMEM((_D, _D), jnp.float32),
            pltpu.VMEM((1, _D), jnp.float32),
        ],
    )(aggp, degp, W2, Wl1, b2r, bl1r, Wl2, bl2r)


def kernel(x, adj, W1, b1, W2, b2, Wl1, bl1, Wl2, bl2):
    dst = adj[1]
    pad = jnp.full((2, _EP - _E), _NP - 1, jnp.int32)
    e5 = (jnp.concatenate([adj, pad], axis=1)
          .reshape(2, _NW, _NSEG, _G, _K).transpose(1, 2, 0, 3, 4))
    x_pad = jnp.pad(x, ((0, _NP - _N), (0, 0)))
    degp = _sc_degree(dst)
    xt1 = _tc_pre(x_pad, W1, degp)
    aggp1 = _sc_pass(e5, xt1)
    xt2 = _tc_mid(aggp1, degp, b1.reshape(1, _D))
    aggp2 = _sc_pass(e5, xt2)
    return _tc_final(aggp2, degp, W2, Wl1, b2.reshape(1, _D),
                     bl1.reshape(1, _D), Wl2, bl2.reshape(1, _D))


# speedup vs baseline: 1.0877x; 1.0877x over previous
import functools

import jax
import jax.numpy as jnp
from jax import lax
from jax.experimental import pallas as pl
from jax.experimental.pallas import tpu as pltpu
from jax.experimental.pallas import tpu_sc as plsc

_N = 10000
_E = 320000
_D = 128
_NP = 10240
_NC, _NS = 2, 16
_NW = _NC * _NS
_EPW = _E // _NW
_EPWP = 10240
_EP = _EPWP * _NW
_K = 80
_NCH = _EPWP // _K
_RPT = _NP // _NS
_B = 512
_GRID = _NP // _B

_mesh = plsc.VectorSubcoreMesh(core_axis_name="c", subcore_axis_name="s")


@functools.partial(
    pl.kernel,
    out_type=jax.ShapeDtypeStruct((_NW, _NP), jnp.float32),
    mesh=_mesh,
    scratch_types=[
        pltpu.VMEM((_EPW,), jnp.int32),
        pltpu.VMEM((_NP,), jnp.float32),
    ],
    compiler_params=pltpu.CompilerParams(needs_layout_passes=False),
)
def _sc_degree(dst_hbm, degp_hbm, dbuf, hist):
    wid = lax.axis_index("c") * _NS + lax.axis_index("s")
    pltpu.sync_copy(dst_hbm.at[pl.ds(wid * _EPW, _EPW)], dbuf)
    zeros = jnp.zeros((16,), jnp.float32)

    def zbody(i, _):
        hist[pl.ds(i * 16, 16)] = zeros
        return 0

    lax.fori_loop(0, _NP // 16, zbody, 0)
    ones = jnp.ones((16,), jnp.float32)

    def body(i, _):
        idx = dbuf[pl.ds(i * 16, 16)]
        plsc.addupdate_scatter(hist, [idx], ones)
        return 0

    lax.fori_loop(0, _EPW // 16, body, 0)
    pltpu.sync_copy(hist, degp_hbm.at[wid])


@functools.partial(
    pl.kernel,
    out_type=jax.ShapeDtypeStruct((_NC, _NP, _D), jnp.float32),
    mesh=_mesh,
    scratch_types=[
        pltpu.VMEM((2, 2, _K), jnp.int32),
        pltpu.VMEM((2, _K, _D), jnp.float32),
        pltpu.VMEM_SHARED((_NP, _D), jnp.float32),
        pltpu.SemaphoreType.DMA((2,)),
    ],
)
def _sc_pass(e4_hbm, xt_hbm, aggp_hbm, islot, rows, acc, gsem):
    cid = lax.axis_index("c")
    sid = lax.axis_index("s")
    wid = cid * _NS + sid

    zeros = jnp.zeros((16,), jnp.float32)

    def zb(i, _):
        rows[0, i // 8, pl.ds((i % 8) * 16, 16)] = zeros
        return 0

    lax.fori_loop(0, _K * 8, zb, 0)

    def za(i, _):
        pltpu.sync_copy(rows.at[0], acc.at[pl.ds(sid * _RPT + i * _K, _K)])
        return 0

    lax.fori_loop(0, _RPT // _K, za, 0)
    plsc.subcore_barrier()

    def g_issue(j, b):
        pltpu.async_copy(xt_hbm.at[islot.at[b, 0]], rows.at[b], gsem.at[b])

    def g_wait(j, b):
        pltpu.make_async_copy(xt_hbm.at[islot.at[b, 0]], rows.at[b],
                              gsem.at[b]).wait()

    pltpu.sync_copy(e4_hbm.at[wid, 0], islot.at[0])
    g_issue(0, 0)

    def pair(o, _):
        for b in (0, 1):
            j = 2 * o + b
            nb = 1 - b

            def prefetch():
                pltpu.sync_copy(e4_hbm.at[wid, j + 1], islot.at[nb])
                g_wait(j, b)
                g_issue(j + 1, nb)

            if b == 0:
                prefetch()
            else:
                @pl.when(o < _NCH // 2 - 1)
                def _():
                    prefetch()

                @pl.when(o == _NCH // 2 - 1)
                def _():
                    g_wait(j, b)

            pltpu.sync_copy(rows.at[b], acc.at[islot.at[b, 1]], add=True)
        return 0

    lax.fori_loop(0, _NCH // 2, pair, 0)

    plsc.subcore_barrier()
    pltpu.sync_copy(
        acc.at[pl.ds(sid * _RPT, _RPT)],
        aggp_hbm.at[cid, pl.ds(sid * _RPT, _RPT)],
    )


def _dinv(degp):
    ones = jnp.ones((_NW, 1), jnp.float32)
    deg = lax.dot_general(degp, ones, (((0,), (0,)), ((), ())))
    return lax.rsqrt(jnp.maximum(deg, 1.0))


def _tc_pre_body(x_ref, w1_ref, degp_ref, xt1_ref):
    xw = jnp.dot(x_ref[...], w1_ref[...], preferred_element_type=jnp.float32)
    xt1_ref[...] = xw * _dinv(degp_ref[...])


def _tc_mid_body(aggp_ref, degp_ref, b1_ref, xt2_ref):
    di = _dinv(degp_ref[...])
    h1 = jnp.maximum(di * (aggp_ref[0] + aggp_ref[1]) + b1_ref[...], 0.0)
    xt2_ref[...] = di * h1


def _tc_final_body(aggp_ref, degp_ref, w2_ref, wl1_ref, b2_ref, bl1_ref,
                   wl2_ref, bl2_ref, out_ref, acc, w25, b25):
    i = pl.program_id(0)

    @pl.when(i == 0)
    def _():
        w25[...] = jnp.dot(w2_ref[...], wl1_ref[...],
                           preferred_element_type=jnp.float32)
        b25[...] = jnp.dot(b2_ref[...], wl1_ref[...],
                           preferred_element_type=jnp.float32) + bl1_ref[...]
        acc[...] = jnp.zeros((1, _D), jnp.float32)

    a2 = _dinv(degp_ref[...]) * (aggp_ref[0] + aggp_ref[1])
    h = jnp.maximum(
        jnp.dot(a2, w25[...], preferred_element_type=jnp.float32) + b25[...],
        0.0,
    )
    row = i * _B + lax.broadcasted_iota(jnp.int32, (_B, 1), 0)
    h = jnp.where(row < _N, h, 0.0)
    acc[...] += jnp.sum(h, axis=0, keepdims=True)

    @pl.when(i == _GRID - 1)
    def _():
        g = acc[...] * (1.0 / _N)
        out_ref[...] = jnp.dot(g, wl2_ref[...],
                               preferred_element_type=jnp.float32) + bl2_ref[...]


def _tc_pre(x_pad, W1, degp):
    return pl.pallas_call(
        _tc_pre_body,
        grid=(_GRID,),
        in_specs=[
            pl.BlockSpec((_B, _D), lambda i: (i, 0)),
            pl.BlockSpec((_D, _D), lambda i: (0, 0)),
            pl.BlockSpec((_NW, _B), lambda i: (0, i)),
        ],
        out_specs=pl.BlockSpec((_B, _D), lambda i: (i, 0)),
        out_shape=jax.ShapeDtypeStruct((_NP, _D), jnp.float32),
    )(x_pad, W1, degp)


def _tc_mid(aggp, degp, b1r):
    return pl.pallas_call(
        _tc_mid_body,
        grid=(_GRID,),
        in_specs=[
            pl.BlockSpec((_NC, _B, _D), lambda i: (0, i, 0)),
            pl.BlockSpec((_NW, _B), lambda i: (0, i)),
            pl.BlockSpec((1, _D), lambda i: (0, 0)),
        ],
        out_specs=pl.BlockSpec((_B, _D), lambda i: (i, 0)),
        out_shape=jax.ShapeDtypeStruct((_NP, _D), jnp.float32),
    )(aggp, degp, b1r)


def _tc_final(aggp, degp, W2, Wl1, b2r, bl1r, Wl2, bl2r):
    return pl.pallas_call(
        _tc_final_body,
        grid=(_GRID,),
        in_specs=[
            pl.BlockSpec((_NC, _B, _D), lambda i: (0, i, 0)),
            pl.BlockSpec((_NW, _B), lambda i: (0, i)),
            pl.BlockSpec((_D, _D), lambda i: (0, 0)),
            pl.BlockSpec((_D, _D), lambda i: (0, 0)),
            pl.BlockSpec((1, _D), lambda i: (0, 0)),
            pl.BlockSpec((1, _D), lambda i: (0, 0)),
            pl.BlockSpec((_D, _D), lambda i: (0, 0)),
            pl.BlockSpec((1, _D), lambda i: (0, 0)),
        ],
        out_specs=pl.BlockSpec((1, _D), lambda i: (0, 0)),
        out_shape=jax.ShapeDtypeStruct((1, _D), jnp.float32),
        scratch_shapes=[
            pltpu.VMEM((1, _D), jnp.float32),
            pltpu.VMEM((_D, _D), jnp.float32),
            pltpu.VMEM((1, _D), jnp.float32),
        ],
    )(aggp, degp, W2, Wl1, b2r, bl1r, Wl2, bl2r)


def kernel(x, adj, W1, b1, W2, b2, Wl1, bl1, Wl2, bl2):
    dst = adj[1]
    pad = jnp.full((2, _EP - _E), _NP - 1, jnp.int32)
    e4 = (jnp.concatenate([adj, pad], axis=1)
          .reshape(2, _NW, _NCH, _K).transpose(1, 2, 0, 3))
    x_pad = jnp.pad(x, ((0, _NP - _N), (0, 0)))
    degp = _sc_degree(dst)
    xt1 = _tc_pre(x_pad, W1, degp)
    aggp1 = _sc_pass(e4, xt1)
    xt2 = _tc_mid(aggp1, degp, b1.reshape(1, _D))
    aggp2 = _sc_pass(e4, xt2)
    return _tc_final(aggp2, degp, W2, Wl1, b2.reshape(1, _D),
                     bl1.reshape(1, _D), Wl2, bl2.reshape(1, _D))


# speedup vs baseline: 3.2591x; 2.9964x over previous
import functools

import jax
import jax.numpy as jnp
from jax import lax
from jax.experimental import pallas as pl
from jax.experimental.pallas import tpu as pltpu
from jax.experimental.pallas import tpu_sc as plsc

_N = 10000
_E = 320000
_D = 128
_NP = 10240
_NC, _NS = 2, 16
_NW = _NC * _NS
_EPW = _E // _NW
_EPWP = 10240
_EP = _EPWP * _NW
_K = 80
_NCH = _EPWP // _K
_RPT = _NP // _NS
_B = 512
_GRID = _NP // _B

_mesh = plsc.VectorSubcoreMesh(core_axis_name="c", subcore_axis_name="s")


@functools.partial(
    pl.kernel,
    out_type=jax.ShapeDtypeStruct((_NW, _NP), jnp.float32),
    mesh=_mesh,
    scratch_types=[
        pltpu.VMEM((_EPW,), jnp.int32),
        pltpu.VMEM((_NP,), jnp.float32),
    ],
    compiler_params=pltpu.CompilerParams(needs_layout_passes=False),
)
def _sc_degree(dst_hbm, degp_hbm, dbuf, hist):
    wid = lax.axis_index("c") * _NS + lax.axis_index("s")
    pltpu.sync_copy(dst_hbm.at[pl.ds(wid * _EPW, _EPW)], dbuf)
    zeros = jnp.zeros((16,), jnp.float32)

    def zbody(i, _):
        hist[pl.ds(i * 16, 16)] = zeros
        return 0

    lax.fori_loop(0, _NP // 16, zbody, 0)
    ones = jnp.ones((16,), jnp.float32)

    def body(i, _):
        idx = dbuf[pl.ds(i * 16, 16)]
        plsc.addupdate_scatter(hist, [idx], ones)
        return 0

    lax.fori_loop(0, _EPW // 16, body, 0)
    pltpu.sync_copy(hist, degp_hbm.at[wid])


@functools.partial(
    pl.kernel,
    out_type=jax.ShapeDtypeStruct((_NC, _NP, _D), jnp.float32),
    mesh=_mesh,
    scratch_types=[
        pltpu.VMEM((2, 2, _K), jnp.int32),
        pltpu.VMEM((2, _K, _D), jnp.float32),
        pltpu.VMEM_SHARED((_NP, _D), jnp.float32),
        pltpu.SemaphoreType.DMA((2,)),
    ],
)
def _sc_pass(e4_hbm, xt_hbm, aggp_hbm, islot, rows, acc, gsem):
    cid = lax.axis_index("c")
    sid = lax.axis_index("s")
    wid = cid * _NS + sid

    zeros = jnp.zeros((16,), jnp.float32)

    def zb(i, _):
        rows[0, i // 8, pl.ds((i % 8) * 16, 16)] = zeros
        return 0

    lax.fori_loop(0, _K * 8, zb, 0)

    def za(i, _):
        pltpu.sync_copy(rows.at[0], acc.at[pl.ds(sid * _RPT + i * _K, _K)])
        return 0

    lax.fori_loop(0, _RPT // _K, za, 0)
    plsc.subcore_barrier()

    def g_issue(j, b):
        pltpu.async_copy(xt_hbm.at[islot.at[b, 0]], rows.at[b], gsem.at[b])

    def g_wait(j, b):
        pltpu.make_async_copy(xt_hbm.at[islot.at[b, 0]], rows.at[b],
                              gsem.at[b]).wait()

    pltpu.sync_copy(e4_hbm.at[wid, 0], islot.at[0])
    g_issue(0, 0)

    def pair(o, _):
        for b in (0, 1):
            j = 2 * o + b
            nb = 1 - b

            def prefetch():
                pltpu.sync_copy(e4_hbm.at[wid, j + 1], islot.at[nb])
                g_wait(j, b)
                g_issue(j + 1, nb)

            if b == 0:
                prefetch()
            else:
                @pl.when(o < _NCH // 2 - 1)
                def _():
                    prefetch()

                @pl.when(o == _NCH // 2 - 1)
                def _():
                    g_wait(j, b)

            pltpu.sync_copy(rows.at[b], acc.at[islot.at[b, 1]], add=True)
        return 0

    lax.fori_loop(0, _NCH // 2, pair, 0)

    plsc.subcore_barrier()
    pltpu.sync_copy(
        acc.at[pl.ds(sid * _RPT, _RPT)],
        aggp_hbm.at[cid, pl.ds(sid * _RPT, _RPT)],
    )


def _dinv(degp):
    ones = jnp.ones((_NW, 1), jnp.float32)
    deg = lax.dot_general(degp, ones, (((0,), (0,)), ((), ())))
    return lax.rsqrt(jnp.maximum(deg, 1.0))


def _tc_pre_body(x_ref, w1_ref, degp_ref, xt1_ref):
    xw = jnp.dot(x_ref[...], w1_ref[...], preferred_element_type=jnp.float32)
    xt1_ref[...] = xw * _dinv(degp_ref[...])


def _tc_mid_body(aggp_ref, degp_ref, b1_ref, xt2_ref):
    di = _dinv(degp_ref[...])
    h1 = jnp.maximum(di * (aggp_ref[0] + aggp_ref[1]) + b1_ref[...], 0.0)
    xt2_ref[...] = di * h1


def _tc_final_body(aggp_ref, degp_ref, w2_ref, wl1_ref, b2_ref, bl1_ref,
                   wl2_ref, bl2_ref, out_ref, acc, w25, b25):
    i = pl.program_id(0)

    @pl.when(i == 0)
    def _():
        w25[...] = jnp.dot(w2_ref[...], wl1_ref[...],
                           preferred_element_type=jnp.float32)
        b25[...] = jnp.dot(b2_ref[...], wl1_ref[...],
                           preferred_element_type=jnp.float32) + bl1_ref[...]
        acc[...] = jnp.zeros((1, _D), jnp.float32)

    a2 = _dinv(degp_ref[...]) * (aggp_ref[0] + aggp_ref[1])
    h = jnp.maximum(
        jnp.dot(a2, w25[...], preferred_element_type=jnp.float32) + b25[...],
        0.0,
    )
    row = i * _B + lax.broadcasted_iota(jnp.int32, (_B, 1), 0)
    h = jnp.where(row < _N, h, 0.0)
    acc[...] += jnp.sum(h, axis=0, keepdims=True)

    @pl.when(i == _GRID - 1)
    def _():
        g = acc[...] * (1.0 / _N)
        out_ref[...] = jnp.dot(g, wl2_ref[...],
                               preferred_element_type=jnp.float32) + bl2_ref[...]


def _tc_pre(x_pad, W1, degp):
    return pl.pallas_call(
        _tc_pre_body,
        grid=(_GRID,),
        in_specs=[
            pl.BlockSpec((_B, _D), lambda i: (i, 0)),
            pl.BlockSpec((_D, _D), lambda i: (0, 0)),
            pl.BlockSpec((_NW, _B), lambda i: (0, i)),
        ],
        out_specs=pl.BlockSpec((_B, _D), lambda i: (i, 0)),
        out_shape=jax.ShapeDtypeStruct((_NP, _D), jnp.float32),
    )(x_pad, W1, degp)


def _tc_mid(aggp, degp, b1r):
    return pl.pallas_call(
        _tc_mid_body,
        grid=(_GRID,),
        in_specs=[
            pl.BlockSpec((_NC, _B, _D), lambda i: (0, i, 0)),
            pl.BlockSpec((_NW, _B), lambda i: (0, i)),
            pl.BlockSpec((1, _D), lambda i: (0, 0)),
        ],
        out_specs=pl.BlockSpec((_B, _D), lambda i: (i, 0)),
        out_shape=jax.ShapeDtypeStruct((_NP, _D), jnp.float32),
    )(aggp, degp, b1r)


def _tc_final(aggp, degp, W2, Wl1, b2r, bl1r, Wl2, bl2r):
    return pl.pallas_call(
        _tc_final_body,
        grid=(_GRID,),
        in_specs=[
            pl.BlockSpec((_NC, _B, _D), lambda i: (0, i, 0)),
            pl.BlockSpec((_NW, _B), lambda i: (0, i)),
            pl.BlockSpec((_D, _D), lambda i: (0, 0)),
            pl.BlockSpec((_D, _D), lambda i: (0, 0)),
            pl.BlockSpec((1, _D), lambda i: (0, 0)),
            pl.BlockSpec((1, _D), lambda i: (0, 0)),
            pl.BlockSpec((_D, _D), lambda i: (0, 0)),
            pl.BlockSpec((1, _D), lambda i: (0, 0)),
        ],
        out_specs=pl.BlockSpec((1, _D), lambda i: (0, 0)),
        out_shape=jax.ShapeDtypeStruct((1, _D), jnp.float32),
        scratch_shapes=[
            pltpu.VMEM((1, _D), jnp.float32),
            pltpu.VMEM((_D, _D), jnp.float32),
            pltpu.VMEM((1, _D), jnp.float32),
        ],
    )(aggp, degp, W2, Wl1, b2r, bl1r, Wl2, bl2r)


def kernel(x, adj, W1, b1, W2, b2, Wl1, bl1, Wl2, bl2):
    dst = adj[1]
    pad = jnp.broadcast_to(
        _N + jnp.arange(_EP - _E, dtype=jnp.int32) % (_NP - _N),
        (2, _EP - _E))
    e4 = (jnp.concatenate([adj, pad], axis=1)
          .reshape(2, _NW, _NCH, _K).transpose(1, 2, 0, 3))
    x_pad = jnp.pad(x, ((0, _NP - _N), (0, 0)))
    degp = _sc_degree(dst)
    xt1 = _tc_pre(x_pad, W1, degp)
    aggp1 = _sc_pass(e4, xt1)
    xt2 = _tc_mid(aggp1, degp, b1.reshape(1, _D))
    aggp2 = _sc_pass(e4, xt2)
    return _tc_final(aggp2, degp, W2, Wl1, b2.reshape(1, _D),
                     bl1.reshape(1, _D), Wl2, bl2.reshape(1, _D))


# speedup vs baseline: 3.2651x; 1.0018x over previous
import functools

import jax
import jax.numpy as jnp
from jax import lax
from jax.experimental import pallas as pl
from jax.experimental.pallas import tpu as pltpu
from jax.experimental.pallas import tpu_sc as plsc

_N = 10000
_E = 320000
_D = 128
_NP = 10240
_NC, _NS = 2, 16
_NW = _NC * _NS
_EPW = _E // _NW
_EPWP = 10240
_EP = _EPWP * _NW
_K = 80
_NCH = _EPWP // _K
_RPT = _NP // _NS
_B = 512
_GRID = _NP // _B

_mesh = plsc.VectorSubcoreMesh(core_axis_name="c", subcore_axis_name="s")


@functools.partial(
    pl.kernel,
    out_type=jax.ShapeDtypeStruct((_NW, _NP), jnp.float32),
    mesh=_mesh,
    scratch_types=[
        pltpu.VMEM((_EPW,), jnp.int32),
        pltpu.VMEM((_NP,), jnp.float32),
    ],
    compiler_params=pltpu.CompilerParams(needs_layout_passes=False),
)
def _sc_degree(dst_hbm, degp_hbm, dbuf, hist):
    wid = lax.axis_index("c") * _NS + lax.axis_index("s")
    pltpu.sync_copy(dst_hbm.at[pl.ds(wid * _EPW, _EPW)], dbuf)
    zeros = jnp.zeros((16,), jnp.float32)

    def zbody(i, _):
        hist[pl.ds(i * 16, 16)] = zeros
        return 0

    lax.fori_loop(0, _NP // 16, zbody, 0)
    ones = jnp.ones((16,), jnp.float32)

    def body(i, _):
        idx = dbuf[pl.ds(i * 16, 16)]
        plsc.addupdate_scatter(hist, [idx], ones)
        return 0

    lax.fori_loop(0, _EPW // 16, body, 0)
    pltpu.sync_copy(hist, degp_hbm.at[wid])


@functools.partial(
    pl.kernel,
    out_type=jax.ShapeDtypeStruct((_NC, _NP, _D), jnp.float32),
    mesh=_mesh,
    scratch_types=[
        pltpu.VMEM((4, 2, _K), jnp.int32),
        pltpu.VMEM((2, _K, _D), jnp.float32),
        pltpu.VMEM_SHARED((_NP, _D), jnp.float32),
        pltpu.SemaphoreType.DMA((2,)),
        pltpu.SemaphoreType.DMA((2,)),
    ],
)
def _sc_pass(e4_hbm, xt_hbm, aggp_hbm, islot, rows, acc, gsem, ssem):
    cid = lax.axis_index("c")
    sid = lax.axis_index("s")
    wid = cid * _NS + sid

    zeros = jnp.zeros((16,), jnp.float32)

    def zb(i, _):
        rows[0, i // 8, pl.ds((i % 8) * 16, 16)] = zeros
        return 0

    lax.fori_loop(0, _K * 8, zb, 0)

    def za(i, _):
        pltpu.sync_copy(rows.at[0], acc.at[pl.ds(sid * _RPT + i * _K, _K)])
        return 0

    lax.fori_loop(0, _RPT // _K, za, 0)
    plsc.subcore_barrier()

    def g_issue(t, b):
        pltpu.async_copy(xt_hbm.at[islot.at[t, 0]], rows.at[b], gsem.at[b])

    def g_wait(t, b):
        pltpu.make_async_copy(xt_hbm.at[islot.at[t, 0]], rows.at[b],
                              gsem.at[b]).wait()

    def s_issue(t, b):
        pltpu.async_copy(rows.at[b], acc.at[islot.at[t, 1]], ssem.at[b],
                         add=True)

    def s_wait(t, b):
        pltpu.make_async_copy(rows.at[b], acc.at[islot.at[t, 1]],
                              ssem.at[b]).wait()

    pltpu.sync_copy(e4_hbm.at[wid, 0], islot.at[0])
    g_issue(0, 0)

    def quad(o, _):
        for u in range(4):
            j = 4 * o + u
            t, b = u, u % 2
            tn, nb = (u + 1) % 4, 1 - b
            tp = (u - 1) % 4

            def prefetch(first):
                pltpu.sync_copy(e4_hbm.at[wid, j + 1], islot.at[tn])
                g_wait(t, b)
                if not first:
                    s_wait(tp, nb)
                g_issue(tn, nb)

            if u == 0:
                @pl.when(o == 0)
                def _():
                    prefetch(True)

                @pl.when(o > 0)
                def _():
                    prefetch(False)
            elif u < 3:
                prefetch(False)
            else:
                @pl.when(o < _NCH // 4 - 1)
                def _():
                    prefetch(False)

                @pl.when(o == _NCH // 4 - 1)
                def _():
                    g_wait(t, b)
                    s_wait(tp, nb)

            s_issue(t, b)
        return 0

    lax.fori_loop(0, _NCH // 4, quad, 0)
    s_wait(3, 1)
    plsc.subcore_barrier()
    pltpu.sync_copy(
        acc.at[pl.ds(sid * _RPT, _RPT)],
        aggp_hbm.at[cid, pl.ds(sid * _RPT, _RPT)],
    )


def _dinv(degp):
    ones = jnp.ones((_NW, 1), jnp.float32)
    deg = lax.dot_general(degp, ones, (((0,), (0,)), ((), ())))
    return lax.rsqrt(jnp.maximum(deg, 1.0))


def _tc_pre_body(x_ref, w1_ref, degp_ref, xt1_ref):
    xw = jnp.dot(x_ref[...], w1_ref[...], preferred_element_type=jnp.float32)
    xt1_ref[...] = xw * _dinv(degp_ref[...])


def _tc_mid_body(aggp_ref, degp_ref, b1_ref, xt2_ref):
    di = _dinv(degp_ref[...])
    h1 = jnp.maximum(di * (aggp_ref[0] + aggp_ref[1]) + b1_ref[...], 0.0)
    xt2_ref[...] = di * h1


def _tc_final_body(aggp_ref, degp_ref, w2_ref, wl1_ref, b2_ref, bl1_ref,
                   wl2_ref, bl2_ref, out_ref, acc, w25, b25):
    i = pl.program_id(0)

    @pl.when(i == 0)
    def _():
        w25[...] = jnp.dot(w2_ref[...], wl1_ref[...],
                           preferred_element_type=jnp.float32)
        b25[...] = jnp.dot(b2_ref[...], wl1_ref[...],
                           preferred_element_type=jnp.float32) + bl1_ref[...]
        acc[...] = jnp.zeros((1, _D), jnp.float32)

    a2 = _dinv(degp_ref[...]) * (aggp_ref[0] + aggp_ref[1])
    h = jnp.maximum(
        jnp.dot(a2, w25[...], preferred_element_type=jnp.float32) + b25[...],
        0.0,
    )
    row = i * _B + lax.broadcasted_iota(jnp.int32, (_B, 1), 0)
    h = jnp.where(row < _N, h, 0.0)
    acc[...] += jnp.sum(h, axis=0, keepdims=True)

    @pl.when(i == _GRID - 1)
    def _():
        g = acc[...] * (1.0 / _N)
        out_ref[...] = jnp.dot(g, wl2_ref[...],
                               preferred_element_type=jnp.float32) + bl2_ref[...]


def _tc_pre(x_pad, W1, degp):
    return pl.pallas_call(
        _tc_pre_body,
        grid=(_GRID,),
        in_specs=[
            pl.BlockSpec((_B, _D), lambda i: (i, 0)),
            pl.BlockSpec((_D, _D), lambda i: (0, 0)),
            pl.BlockSpec((_NW, _B), lambda i: (0, i)),
        ],
        out_specs=pl.BlockSpec((_B, _D), lambda i: (i, 0)),
        out_shape=jax.ShapeDtypeStruct((_NP, _D), jnp.float32),
    )(x_pad, W1, degp)


def _tc_mid(aggp, degp, b1r):
    return pl.pallas_call(
        _tc_mid_body,
        grid=(_GRID,),
        in_specs=[
            pl.BlockSpec((_NC, _B, _D), lambda i: (0, i, 0)),
            pl.BlockSpec((_NW, _B), lambda i: (0, i)),
            pl.BlockSpec((1, _D), lambda i: (0, 0)),
        ],
        out_specs=pl.BlockSpec((_B, _D), lambda i: (i, 0)),
        out_shape=jax.ShapeDtypeStruct((_NP, _D), jnp.float32),
    )(aggp, degp, b1r)


def _tc_final(aggp, degp, W2, Wl1, b2r, bl1r, Wl2, bl2r):
    return pl.pallas_call(
        _tc_final_body,
        grid=(_GRID,),
        in_specs=[
            pl.BlockSpec((_NC, _B, _D), lambda i: (0, i, 0)),
            pl.BlockSpec((_NW, _B), lambda i: (0, i)),
            pl.BlockSpec((_D, _D), lambda i: (0, 0)),
            pl.BlockSpec((_D, _D), lambda i: (0, 0)),
            pl.BlockSpec((1, _D), lambda i: (0, 0)),
            pl.BlockSpec((1, _D), lambda i: (0, 0)),
            pl.BlockSpec((_D, _D), lambda i: (0, 0)),
            pl.BlockSpec((1, _D), lambda i: (0, 0)),
        ],
        out_specs=pl.BlockSpec((1, _D), lambda i: (0, 0)),
        out_shape=jax.ShapeDtypeStruct((1, _D), jnp.float32),
        scratch_shapes=[
            pltpu.VMEM((1, _D), jnp.float32),
            pltpu.VMEM((_D, _D), jnp.float32),
            pltpu.VMEM((1, _D), jnp.float32),
        ],
    )(aggp, degp, W2, Wl1, b2r, bl1r, Wl2, bl2r)


def kernel(x, adj, W1, b1, W2, b2, Wl1, bl1, Wl2, bl2):
    dst = adj[1]
    pad = jnp.broadcast_to(
        _N + jnp.arange(_EP - _E, dtype=jnp.int32) % (_NP - _N),
        (2, _EP - _E))
    e4 = (jnp.concatenate([adj, pad], axis=1)
          .reshape(2, _NW, _NCH, _K).transpose(1, 2, 0, 3))
    x_pad = jnp.pad(x, ((0, _NP - _N), (0, 0)))
    degp = _sc_degree(dst)
    xt1 = _tc_pre(x_pad, W1, degp)
    aggp1 = _sc_pass(e4, xt1)
    xt2 = _tc_mid(aggp1, degp, b1.reshape(1, _D))
    aggp2 = _sc_pass(e4, xt2)
    return _tc_final(aggp2, degp, W2, Wl1, b2.reshape(1, _D),
                     bl1.reshape(1, _D), Wl2, bl2.reshape(1, _D))


# speedup vs baseline: 3.2700x; 1.0015x over previous
import functools

import jax
import jax.numpy as jnp
from jax import lax
from jax.experimental import pallas as pl
from jax.experimental.pallas import tpu as pltpu
from jax.experimental.pallas import tpu_sc as plsc

_N = 10000
_E = 320000
_D = 128
_NP = 10240
_NC, _NS = 2, 16
_NW = _NC * _NS
_EPW = _E // _NW
_EPWP = 10240
_EP = _EPWP * _NW
_K = 80
_NCH = _EPWP // _K
_RPT = _NP // _NS
_B = 512
_GRID = _NP // _B

_mesh = plsc.VectorSubcoreMesh(core_axis_name="c", subcore_axis_name="s")


@functools.partial(
    pl.kernel,
    out_type=jax.ShapeDtypeStruct((_NW, _NP), jnp.float32),
    mesh=_mesh,
    scratch_types=[
        pltpu.VMEM((_EPW,), jnp.int32),
        pltpu.VMEM((_NP,), jnp.float32),
    ],
    compiler_params=pltpu.CompilerParams(needs_layout_passes=False),
)
def _sc_degree(dst_hbm, degp_hbm, dbuf, hist):
    wid = lax.axis_index("c") * _NS + lax.axis_index("s")
    pltpu.sync_copy(dst_hbm.at[pl.ds(wid * _EPW, _EPW)], dbuf)
    zeros = jnp.zeros((16,), jnp.float32)

    def zbody(i, _):
        hist[pl.ds(i * 16, 16)] = zeros
        return 0

    lax.fori_loop(0, _NP // 16, zbody, 0)
    ones = jnp.ones((16,), jnp.float32)

    def body(i, _):
        idx = dbuf[pl.ds(i * 16, 16)]
        plsc.addupdate_scatter(hist, [idx], ones)
        return 0

    lax.fori_loop(0, _EPW // 16, body, 0)
    pltpu.sync_copy(hist, degp_hbm.at[wid])


@functools.partial(
    pl.kernel,
    out_type=jax.ShapeDtypeStruct((_NC, _NP, _D), jnp.float32),
    mesh=_mesh,
    scratch_types=[
        pltpu.VMEM((4, 2, _K), jnp.int32),
        pltpu.VMEM((2, _K, _D), jnp.float32),
        pltpu.VMEM_SHARED((_NP, _D), jnp.float32),
        pltpu.SemaphoreType.DMA((2,)),
        pltpu.SemaphoreType.DMA((2,)),
        pltpu.SemaphoreType.DMA((4,)),
    ],
)
def _sc_pass(e4_hbm, xt_hbm, aggp_hbm, islot, rows, acc, gsem, ssem, isem):
    cid = lax.axis_index("c")
    sid = lax.axis_index("s")
    wid = cid * _NS + sid

    zeros = jnp.zeros((16,), jnp.float32)

    def zb(i, _):
        rows[0, i // 8, pl.ds((i % 8) * 16, 16)] = zeros
        return 0

    lax.fori_loop(0, _K * 8, zb, 0)

    def za(i, _):
        pltpu.sync_copy(rows.at[0], acc.at[pl.ds(sid * _RPT + i * _K, _K)])
        return 0

    lax.fori_loop(0, _RPT // _K, za, 0)
    plsc.subcore_barrier()

    def g_issue(t, b):
        pltpu.async_copy(xt_hbm.at[islot.at[t, 0]], rows.at[b], gsem.at[b])

    def g_wait(t, b):
        pltpu.make_async_copy(xt_hbm.at[islot.at[t, 0]], rows.at[b],
                              gsem.at[b]).wait()

    def s_issue(t, b):
        pltpu.async_copy(rows.at[b], acc.at[islot.at[t, 1]], ssem.at[b],
                         add=True)

    def s_wait(t, b):
        pltpu.make_async_copy(rows.at[b], acc.at[islot.at[t, 1]],
                              ssem.at[b]).wait()

    def i_issue(j, t):
        pltpu.async_copy(e4_hbm.at[wid, j], islot.at[t], isem.at[t])

    def i_wait(j, t):
        pltpu.make_async_copy(e4_hbm.at[wid, j], islot.at[t],
                              isem.at[t]).wait()

    pltpu.sync_copy(e4_hbm.at[wid, 0], islot.at[0])
    i_issue(1, 1)
    g_issue(0, 0)

    def quad(o, _):
        for u in range(4):
            j = 4 * o + u
            t, b = u, u % 2
            tn, nb = (u + 1) % 4, 1 - b
            tp, tq = (u - 1) % 4, (u + 2) % 4

            if u < 2:
                i_issue(j + 2, tq)
            else:
                @pl.when(o < _NCH // 4 - 1)
                def _():
                    i_issue(j + 2, tq)

            def prefetch(first):
                i_wait(j + 1, tn)
                g_wait(t, b)
                if not first:
                    s_wait(tp, nb)
                g_issue(tn, nb)

            if u == 0:
                @pl.when(o == 0)
                def _():
                    prefetch(True)

                @pl.when(o > 0)
                def _():
                    prefetch(False)
            elif u < 3:
                prefetch(False)
            else:
                @pl.when(o < _NCH // 4 - 1)
                def _():
                    prefetch(False)

                @pl.when(o == _NCH // 4 - 1)
                def _():
                    g_wait(t, b)
                    s_wait(tp, nb)

            s_issue(t, b)
        return 0

    lax.fori_loop(0, _NCH // 4, quad, 0)
    s_wait(3, 1)
    plsc.subcore_barrier()
    pltpu.sync_copy(
        acc.at[pl.ds(sid * _RPT, _RPT)],
        aggp_hbm.at[cid, pl.ds(sid * _RPT, _RPT)],
    )


def _dinv(degp):
    ones = jnp.ones((_NW, 1), jnp.float32)
    deg = lax.dot_general(degp, ones, (((0,), (0,)), ((), ())))
    return lax.rsqrt(jnp.maximum(deg, 1.0))


def _tc_pre_body(x_ref, w1_ref, degp_ref, xt1_ref):
    xw = jnp.dot(x_ref[...], w1_ref[...], preferred_element_type=jnp.float32)
    xt1_ref[...] = xw * _dinv(degp_ref[...])


def _tc_mid_body(aggp_ref, degp_ref, b1_ref, xt2_ref):
    di = _dinv(degp_ref[...])
    h1 = jnp.maximum(di * (aggp_ref[0] + aggp_ref[1]) + b1_ref[...], 0.0)
    xt2_ref[...] = di * h1


def _tc_final_body(aggp_ref, degp_ref, w2_ref, wl1_ref, b2_ref, bl1_ref,
                   wl2_ref, bl2_ref, out_ref, acc, w25, b25):
    i = pl.program_id(0)

    @pl.when(i == 0)
    def _():
        w25[...] = jnp.dot(w2_ref[...], wl1_ref[...],
                           preferred_element_type=jnp.float32)
        b25[...] = jnp.dot(b2_ref[...], wl1_ref[...],
                           preferred_element_type=jnp.float32) + bl1_ref[...]
        acc[...] = jnp.zeros((1, _D), jnp.float32)

    a2 = _dinv(degp_ref[...]) * (aggp_ref[0] + aggp_ref[1])
    h = jnp.maximum(
        jnp.dot(a2, w25[...], preferred_element_type=jnp.float32) + b25[...],
        0.0,
    )
    row = i * _B + lax.broadcasted_iota(jnp.int32, (_B, 1), 0)
    h = jnp.where(row < _N, h, 0.0)
    acc[...] += jnp.sum(h, axis=0, keepdims=True)

    @pl.when(i == _GRID - 1)
    def _():
        g = acc[...] * (1.0 / _N)
        out_ref[...] = jnp.dot(g, wl2_ref[...],
                               preferred_element_type=jnp.float32) + bl2_ref[...]


def _tc_pre(x_pad, W1, degp):
    return pl.pallas_call(
        _tc_pre_body,
        grid=(_GRID,),
        in_specs=[
            pl.BlockSpec((_B, _D), lambda i: (i, 0)),
            pl.BlockSpec((_D, _D), lambda i: (0, 0)),
            pl.BlockSpec((_NW, _B), lambda i: (0, i)),
        ],
        out_specs=pl.BlockSpec((_B, _D), lambda i: (i, 0)),
        out_shape=jax.ShapeDtypeStruct((_NP, _D), jnp.float32),
    )(x_pad, W1, degp)


def _tc_mid(aggp, degp, b1r):
    return pl.pallas_call(
        _tc_mid_body,
        grid=(_GRID,),
        in_specs=[
            pl.BlockSpec((_NC, _B, _D), lambda i: (0, i, 0)),
            pl.BlockSpec((_NW, _B), lambda i: (0, i)),
            pl.BlockSpec((1, _D), lambda i: (0, 0)),
        ],
        out_specs=pl.BlockSpec((_B, _D), lambda i: (i, 0)),
        out_shape=jax.ShapeDtypeStruct((_NP, _D), jnp.float32),
    )(aggp, degp, b1r)


def _tc_final(aggp, degp, W2, Wl1, b2r, bl1r, Wl2, bl2r):
    return pl.pallas_call(
        _tc_final_body,
        grid=(_GRID,),
        in_specs=[
            pl.BlockSpec((_NC, _B, _D), lambda i: (0, i, 0)),
            pl.BlockSpec((_NW, _B), lambda i: (0, i)),
            pl.BlockSpec((_D, _D), lambda i: (0, 0)),
            pl.BlockSpec((_D, _D), lambda i: (0, 0)),
            pl.BlockSpec((1, _D), lambda i: (0, 0)),
            pl.BlockSpec((1, _D), lambda i: (0, 0)),
            pl.BlockSpec((_D, _D), lambda i: (0, 0)),
            pl.BlockSpec((1, _D), lambda i: (0, 0)),
        ],
        out_specs=pl.BlockSpec((1, _D), lambda i: (0, 0)),
        out_shape=jax.ShapeDtypeStruct((1, _D), jnp.float32),
        scratch_shapes=[
            pltpu.VMEM((1, _D), jnp.float32),
            pltpu.VMEM((_D, _D), jnp.float32),
            pltpu.VMEM((1, _D), jnp.float32),
        ],
    )(aggp, degp, W2, Wl1, b2r, bl1r, Wl2, bl2r)


def kernel(x, adj, W1, b1, W2, b2, Wl1, bl1, Wl2, bl2):
    dst = adj[1]
    pad = jnp.broadcast_to(
        _N + jnp.arange(_EP - _E, dtype=jnp.int32) % (_NP - _N),
        (2, _EP - _E))
    e4 = (jnp.concatenate([adj, pad], axis=1)
          .reshape(2, _NW, _NCH, _K).transpose(1, 2, 0, 3))
    x_pad = jnp.pad(x, ((0, _NP - _N), (0, 0)))
    degp = _sc_degree(dst)
    xt1 = _tc_pre(x_pad, W1, degp)
    aggp1 = _sc_pass(e4, xt1)
    xt2 = _tc_mid(aggp1, degp, b1.reshape(1, _D))
    aggp2 = _sc_pass(e4, xt2)
    return _tc_final(aggp2, degp, W2, Wl1, b2.reshape(1, _D),
                     bl1.reshape(1, _D), Wl2, bl2.reshape(1, _D))


# speedup vs baseline: 3.3616x; 1.0280x over previous
import functools

import jax
import jax.numpy as jnp
from jax import lax
from jax.experimental import pallas as pl
from jax.experimental.pallas import tpu as pltpu
from jax.experimental.pallas import tpu_sc as plsc

_N = 10000
_E = 320000
_D = 128
_NP = 10240
_NC, _NS = 2, 16
_NW = _NC * _NS
_EPW = _E // _NW
_EPWP = 10240
_EP = _EPWP * _NW
_K = 80
_NCH = _EPWP // _K
_RPT = _NP // _NS
_B = 512
_GRID = _NP // _B

_mesh = plsc.VectorSubcoreMesh(core_axis_name="c", subcore_axis_name="s")


@functools.partial(
    pl.kernel,
    out_type=jax.ShapeDtypeStruct((_NW, _NP), jnp.float32),
    mesh=_mesh,
    scratch_types=[
        pltpu.VMEM((_EPW,), jnp.int32),
        pltpu.VMEM((_NP,), jnp.float32),
    ],
    compiler_params=pltpu.CompilerParams(needs_layout_passes=False),
)
def _sc_degree(dst_hbm, degp_hbm, dbuf, hist):
    wid = lax.axis_index("c") * _NS + lax.axis_index("s")
    pltpu.sync_copy(dst_hbm.at[pl.ds(wid * _EPW, _EPW)], dbuf)
    zeros = jnp.zeros((16,), jnp.float32)

    def zbody(i, _):
        hist[pl.ds(i * 16, 16)] = zeros
        return 0

    lax.fori_loop(0, _NP // 16, zbody, 0)
    ones = jnp.ones((16,), jnp.float32)

    def body(i, _):
        idx = dbuf[pl.ds(i * 16, 16)]
        plsc.addupdate_scatter(hist, [idx], ones)
        return 0

    lax.fori_loop(0, _EPW // 16, body, 0)
    pltpu.sync_copy(hist, degp_hbm.at[wid])


@functools.partial(
    pl.kernel,
    out_type=jax.ShapeDtypeStruct((_NC, _NP, _D), jnp.float32),
    mesh=_mesh,
    scratch_types=[
        pltpu.VMEM((4, 2, _K), jnp.int32),
        pltpu.VMEM((2, _K, _D), jnp.float32),
        pltpu.VMEM_SHARED((_NP, _D), jnp.float32),
        pltpu.SemaphoreType.DMA((2,)),
        pltpu.SemaphoreType.DMA((2,)),
        pltpu.SemaphoreType.DMA((4,)),
    ],
)
def _sc_pass(src4_hbm, dst4_hbm, xt_hbm, aggp_hbm, islot, rows, acc, gsem,
             ssem, isem):
    cid = lax.axis_index("c")
    sid = lax.axis_index("s")
    wid = cid * _NS + sid

    zeros = jnp.zeros((16,), jnp.float32)

    def zb(i, _):
        rows[0, i // 8, pl.ds((i % 8) * 16, 16)] = zeros
        return 0

    lax.fori_loop(0, _K * 8, zb, 0)

    def za(i, _):
        pltpu.sync_copy(rows.at[0], acc.at[pl.ds(sid * _RPT + i * _K, _K)])
        return 0

    lax.fori_loop(0, _RPT // _K, za, 0)
    plsc.subcore_barrier()

    def g_issue(t, b):
        pltpu.async_copy(xt_hbm.at[islot.at[t, 0]], rows.at[b], gsem.at[b])

    def g_wait(t, b):
        pltpu.make_async_copy(xt_hbm.at[islot.at[t, 0]], rows.at[b],
                              gsem.at[b]).wait()

    def s_issue(t, b):
        pltpu.async_copy(rows.at[b], acc.at[islot.at[t, 1]], ssem.at[b],
                         add=True)

    def s_wait(t, b):
        pltpu.make_async_copy(rows.at[b], acc.at[islot.at[t, 1]],
                              ssem.at[b]).wait()

    def i_issue(j, t):
        pltpu.async_copy(src4_hbm.at[wid, j], islot.at[t, 0], isem.at[t])
        pltpu.async_copy(dst4_hbm.at[wid, j], islot.at[t, 1], isem.at[t])

    def i_wait(j, t):
        pltpu.make_async_copy(src4_hbm.at[wid, j], islot.at[t, 0],
                              isem.at[t]).wait()
        pltpu.make_async_copy(dst4_hbm.at[wid, j], islot.at[t, 1],
                              isem.at[t]).wait()

    pltpu.sync_copy(src4_hbm.at[wid, 0], islot.at[0, 0])
    pltpu.sync_copy(dst4_hbm.at[wid, 0], islot.at[0, 1])
    i_issue(1, 1)
    g_issue(0, 0)

    def quad(o, _):
        for u in range(4):
            j = 4 * o + u
            t, b = u, u % 2
            tn, nb = (u + 1) % 4, 1 - b
            tp, tq = (u - 1) % 4, (u + 2) % 4

            if u < 2:
                i_issue(j + 2, tq)
            else:
                @pl.when(o < _NCH // 4 - 1)
                def _():
                    i_issue(j + 2, tq)

            def prefetch(first):
                i_wait(j + 1, tn)
                g_wait(t, b)
                if not first:
                    s_wait(tp, nb)
                g_issue(tn, nb)

            if u == 0:
                @pl.when(o == 0)
                def _():
                    prefetch(True)

                @pl.when(o > 0)
                def _():
                    prefetch(False)
            elif u < 3:
                prefetch(False)
            else:
                @pl.when(o < _NCH // 4 - 1)
                def _():
                    prefetch(False)

                @pl.when(o == _NCH // 4 - 1)
                def _():
                    g_wait(t, b)
                    s_wait(tp, nb)

            s_issue(t, b)
        return 0

    lax.fori_loop(0, _NCH // 4, quad, 0)
    s_wait(3, 1)
    plsc.subcore_barrier()
    pltpu.sync_copy(
        acc.at[pl.ds(sid * _RPT, _RPT)],
        aggp_hbm.at[cid, pl.ds(sid * _RPT, _RPT)],
    )


def _dinv(degp):
    ones = jnp.ones((_NW, 1), jnp.float32)
    deg = lax.dot_general(degp, ones, (((0,), (0,)), ((), ())))
    return lax.rsqrt(jnp.maximum(deg, 1.0))


def _tc_pre_body(x_ref, w1_ref, degp_ref, xt1_ref):
    xw = jnp.dot(x_ref[...], w1_ref[...], preferred_element_type=jnp.float32)
    xt1_ref[...] = xw * _dinv(degp_ref[...])


def _tc_mid_body(aggp_ref, degp_ref, b1_ref, xt2_ref):
    di = _dinv(degp_ref[...])
    h1 = jnp.maximum(di * (aggp_ref[0] + aggp_ref[1]) + b1_ref[...], 0.0)
    xt2_ref[...] = di * h1


def _tc_final_body(aggp_ref, degp_ref, w2_ref, wl1_ref, b2_ref, bl1_ref,
                   wl2_ref, bl2_ref, out_ref, acc, w25, b25):
    i = pl.program_id(0)

    @pl.when(i == 0)
    def _():
        w25[...] = jnp.dot(w2_ref[...], wl1_ref[...],
                           preferred_element_type=jnp.float32)
        b25[...] = jnp.dot(b2_ref[...], wl1_ref[...],
                           preferred_element_type=jnp.float32) + bl1_ref[...]
        acc[...] = jnp.zeros((1, _D), jnp.float32)

    a2 = _dinv(degp_ref[...]) * (aggp_ref[0] + aggp_ref[1])
    h = jnp.maximum(
        jnp.dot(a2, w25[...], preferred_element_type=jnp.float32) + b25[...],
        0.0,
    )
    row = i * _B + lax.broadcasted_iota(jnp.int32, (_B, 1), 0)
    h = jnp.where(row < _N, h, 0.0)
    acc[...] += jnp.sum(h, axis=0, keepdims=True)

    @pl.when(i == _GRID - 1)
    def _():
        g = acc[...] * (1.0 / _N)
        out_ref[...] = jnp.dot(g, wl2_ref[...],
                               preferred_element_type=jnp.float32) + bl2_ref[...]


def _tc_pre(x_pad, W1, degp):
    return pl.pallas_call(
        _tc_pre_body,
        grid=(_GRID,),
        in_specs=[
            pl.BlockSpec((_B, _D), lambda i: (i, 0)),
            pl.BlockSpec((_D, _D), lambda i: (0, 0)),
            pl.BlockSpec((_NW, _B), lambda i: (0, i)),
        ],
        out_specs=pl.BlockSpec((_B, _D), lambda i: (i, 0)),
        out_shape=jax.ShapeDtypeStruct((_NP, _D), jnp.float32),
    )(x_pad, W1, degp)


def _tc_mid(aggp, degp, b1r):
    return pl.pallas_call(
        _tc_mid_body,
        grid=(_GRID,),
        in_specs=[
            pl.BlockSpec((_NC, _B, _D), lambda i: (0, i, 0)),
            pl.BlockSpec((_NW, _B), lambda i: (0, i)),
            pl.BlockSpec((1, _D), lambda i: (0, 0)),
        ],
        out_specs=pl.BlockSpec((_B, _D), lambda i: (i, 0)),
        out_shape=jax.ShapeDtypeStruct((_NP, _D), jnp.float32),
    )(aggp, degp, b1r)


def _tc_final(aggp, degp, W2, Wl1, b2r, bl1r, Wl2, bl2r):
    return pl.pallas_call(
        _tc_final_body,
        grid=(_GRID,),
        in_specs=[
            pl.BlockSpec((_NC, _B, _D), lambda i: (0, i, 0)),
            pl.BlockSpec((_NW, _B), lambda i: (0, i)),
            pl.BlockSpec((_D, _D), lambda i: (0, 0)),
            pl.BlockSpec((_D, _D), lambda i: (0, 0)),
            pl.BlockSpec((1, _D), lambda i: (0, 0)),
            pl.BlockSpec((1, _D), lambda i: (0, 0)),
            pl.BlockSpec((_D, _D), lambda i: (0, 0)),
            pl.BlockSpec((1, _D), lambda i: (0, 0)),
        ],
        out_specs=pl.BlockSpec((1, _D), lambda i: (0, 0)),
        out_shape=jax.ShapeDtypeStruct((1, _D), jnp.float32),
        scratch_shapes=[
            pltpu.VMEM((1, _D), jnp.float32),
            pltpu.VMEM((_D, _D), jnp.float32),
            pltpu.VMEM((1, _D), jnp.float32),
        ],
    )(aggp, degp, W2, Wl1, b2r, bl1r, Wl2, bl2r)


def kernel(x, adj, W1, b1, W2, b2, Wl1, bl1, Wl2, bl2):
    dst = adj[1]
    padv = _N + jnp.arange(_EP - _E, dtype=jnp.int32) % (_NP - _N)
    src4 = jnp.concatenate([adj[0], padv]).reshape(_NW, _NCH, _K)
    dst4 = jnp.concatenate([adj[1], padv]).reshape(_NW, _NCH, _K)
    x_pad = jnp.pad(x, ((0, _NP - _N), (0, 0)))
    degp = _sc_degree(dst)
    xt1 = _tc_pre(x_pad, W1, degp)
    aggp1 = _sc_pass(src4, dst4, xt1)
    xt2 = _tc_mid(aggp1, degp, b1.reshape(1, _D))
    aggp2 = _sc_pass(src4, dst4, xt2)
    return _tc_final(aggp2, degp, W2, Wl1, b2.reshape(1, _D),
                     bl1.reshape(1, _D), Wl2, bl2.reshape(1, _D))
